# baseline (speedup 1.0000x reference)
import functools

import jax
import jax.numpy as jnp
from jax import lax
from jax.experimental import pallas as pl
from jax.experimental.pallas import tpu as pltpu
from jax.experimental.pallas import tpu_sc as plsc

R = 512
N = 100000
B = 4096
L = 16
NC = 2
NS = 16
NW = NC * NS
RPW = R // NW
NCHUNK = B // 128

_HBM = pl.BlockSpec(memory_space=pltpu.MemorySpace.HBM)


def _copy_body(t_ref, o_ref, sem):
    pltpu.make_async_copy(t_ref, o_ref, sem).start()
    pltpu.make_async_copy(t_ref, o_ref, sem).wait()


_mesh = plsc.VectorSubcoreMesh(
    core_axis_name="c", subcore_axis_name="s", num_cores=NC, num_subcores=NS)


@functools.partial(
    pl.kernel,
    mesh=_mesh,
    compiler_params=pltpu.CompilerParams(needs_layout_passes=False),
    scratch_types=[
        pltpu.VMEM((B,), jnp.int32),
        pltpu.VMEM((N,), jnp.int32),
        pltpu.VMEM((B,), jnp.int32),
        pltpu.VMEM((B,), jnp.float32),
        pltpu.VMEM((NCHUNK, 128), jnp.int32),
        pltpu.VMEM((NCHUNK, 128), jnp.float32),
        pltpu.SemaphoreType.DMA,
    ],
)
def _sc_scatter(idx_hbm, v_hbm, out_hbm,
                idx_v, m_v, winj_v, vrow_v, addr_v, val_v, sem):
    wid = lax.axis_index("s") * NC + lax.axis_index("c")
    pltpu.sync_copy(idx_hbm, idx_v)
    lane = lax.iota(jnp.int32, L)

    def _mb(k, _):
        idxc = idx_v[pl.ds(k * L, L)]
        jvec = k * L + lane
        plsc.store_scatter(m_v, [idxc], jvec)

        def _fix(i, __):
            w = plsc.load_gather(m_v, [idxc])
            plsc.store_scatter(m_v, [idxc], jvec, mask=jvec > w)
            return 0

        lax.fori_loop(0, 15, _fix, 0)
        return 0

    lax.fori_loop(0, B // L, _mb, 0)

    def _wj(k, _):
        idxc = idx_v[pl.ds(k * L, L)]
        winj_v[pl.ds(k * L, L)] = plsc.load_gather(m_v, [idxc])
        return 0

    lax.fori_loop(0, B // L, _wj, 0)

    def _row(r, _):
        rg = wid * RPW + r
        pltpu.sync_copy(v_hbm.at[rg], vrow_v)
        base = rg * N
        handles = []
        for c in range(NCHUNK):
            def _sub(l, __, c=c):
                s = c * 128 + l * L
                idxc = idx_v[pl.ds(s, L)]
                addr_v[c, pl.ds(l * L, L)] = idxc + base
                w = winj_v[pl.ds(s, L)]
                val_v[c, pl.ds(l * L, L)] = plsc.load_gather(vrow_v, [w])
                return 0

            lax.fori_loop(0, 128 // L, _sub, 0)
            handles.append(
                pltpu.async_copy(val_v.at[c], out_hbm.at[addr_v.at[c]], sem))
        for h in handles:
            h.wait()
        return 0

    lax.fori_loop(0, RPW, _row, 0)


def kernel(t, idx, v):
    t1 = t.reshape(R * N)
    v2 = v.reshape(R, B)

    out0 = pl.pallas_call(
        _copy_body,
        out_shape=jax.ShapeDtypeStruct((R * N,), jnp.float32),
        in_specs=[_HBM],
        out_specs=_HBM,
        scratch_shapes=[pltpu.SemaphoreType.DMA],
    )(t1)

    ref = jax.new_ref(out0)
    _sc_scatter(idx, v2, ref)
    return ref[...].reshape(t.shape)


# speedup vs baseline: 3.1740x; 3.1740x over previous
import functools

import jax
import jax.numpy as jnp
from jax import lax
from jax.experimental import pallas as pl
from jax.experimental.pallas import tpu as pltpu
from jax.experimental.pallas import tpu_sc as plsc

R = 512
N = 100000
B = 4096
L = 16
NC = 2
NS = 16
NW = NC * NS
RPW = R // NW
NCHUNK = B // 128

_HBM = pl.BlockSpec(memory_space=pltpu.MemorySpace.HBM)

CROWS = 8


def _copy_body(t_ref, o_ref):
    o_ref[...] = t_ref[...]


_mesh = plsc.VectorSubcoreMesh(
    core_axis_name="c", subcore_axis_name="s", num_cores=NC, num_subcores=NS)


@functools.partial(
    pl.kernel,
    mesh=_mesh,
    compiler_params=pltpu.CompilerParams(needs_layout_passes=False),
    scratch_types=[
        pltpu.VMEM((B,), jnp.int32),
        pltpu.VMEM((N,), jnp.int32),
        pltpu.VMEM((B,), jnp.int32),
        pltpu.VMEM((B,), jnp.float32),
        pltpu.VMEM((B,), jnp.int32),
        pltpu.VMEM((B,), jnp.int32),
        pltpu.VMEM((B,), jnp.float32),
        pltpu.VMEM((B,), jnp.float32),
        pltpu.SemaphoreType.DMA,
    ],
)
def _sc_scatter(idx_hbm, v_hbm, out_hbm,
                idx_v, m_v, winj_v, vrow_v, addr0, addr1, val0, val1, sem):
    addr_b = (addr0, addr1)
    val_b = (val0, val1)
    wid = lax.axis_index("s") * NC + lax.axis_index("c")
    pltpu.sync_copy(idx_hbm, idx_v)
    lane = lax.iota(jnp.int32, L)

    def _mb(k, _):
        idxc = idx_v[pl.ds(k * L, L)]
        jvec = k * L + lane
        plsc.store_scatter(m_v, [idxc], jvec)

        def _fix(i, __):
            w = plsc.load_gather(m_v, [idxc])
            plsc.store_scatter(m_v, [idxc], jvec, mask=jvec > w)
            return 0

        lax.fori_loop(0, 15, _fix, 0)
        return 0

    lax.fori_loop(0, B // L, _mb, 0)

    def _wj(k, _):
        idxc = idx_v[pl.ds(k * L, L)]
        winj_v[pl.ds(k * L, L)] = plsc.load_gather(m_v, [idxc])
        return 0

    lax.fori_loop(0, B // L, _wj, 0)

    def _stage(r, p):
        rg = wid * RPW + r
        pltpu.sync_copy(v_hbm.at[rg], vrow_v)
        base = rg * N

        def _chunk(k, _):
            s = k * L
            idxc = idx_v[pl.ds(s, L)]
            addr_b[p][pl.ds(s, L)] = idxc + base
            w = winj_v[pl.ds(s, L)]
            val_b[p][pl.ds(s, L)] = plsc.load_gather(vrow_v, [w])
            return 0

        lax.fori_loop(0, B // L, _chunk, 0)

    def _fire(p):
        return pltpu.async_copy(val_b[p], out_hbm.at[addr_b[p]], sem)

    _stage(0, 0)
    h = _fire(0)
    for r in range(1, RPW):
        p = r % 2
        _stage(r, p)
        h.wait()
        h = _fire(p)
    h.wait()


def kernel(t, idx, v):
    t2 = t.reshape(R, N)
    v2 = v.reshape(R, B)

    out0 = pl.pallas_call(
        _copy_body,
        grid=(R // CROWS,),
        in_specs=[pl.BlockSpec((CROWS, N), lambda i: (i, 0))],
        out_specs=pl.BlockSpec((CROWS, N), lambda i: (i, 0)),
        out_shape=jax.ShapeDtypeStruct((R, N), jnp.float32),
    )(t2)

    ref = jax.new_ref(out0.reshape(R * N))
    _sc_scatter(idx, v2, ref)
    return ref[...].reshape(t.shape)


# speedup vs baseline: 3.1743x; 1.0001x over previous
import functools

import jax
import jax.numpy as jnp
from jax import lax
from jax.experimental import pallas as pl
from jax.experimental.pallas import tpu as pltpu
from jax.experimental.pallas import tpu_sc as plsc

R = 512
N = 100000
B = 4096
L = 16
NC = 2
NS = 16
NW = NC * NS
RPW = R // NW

_HBM = pl.BlockSpec(memory_space=pltpu.MemorySpace.HBM)

CH = 256000
NCHUNKS = (R * N) // CH
SLOTS = 8
LAG = 4


def _copy_body(t_ref, o_ref, buf, sem_in, sem_out):
    def _in(i, p):
        return pltpu.make_async_copy(
            t_ref.at[pl.ds(i * CH, CH)], buf.at[p], sem_in.at[p])

    def _out(i, p):
        return pltpu.make_async_copy(
            buf.at[p], o_ref.at[pl.ds(i * CH, CH)], sem_out.at[p])

    for i in range(NCHUNKS + LAG):
        if i < NCHUNKS:
            p = i % SLOTS
            if i >= SLOTS:
                _out(i - SLOTS, p).wait()
            _in(i, p).start()
        j = i - LAG
        if 0 <= j < NCHUNKS:
            pj = j % SLOTS
            _in(j, pj).wait()
            _out(j, pj).start()
    for j in range(NCHUNKS - SLOTS, NCHUNKS):
        _out(j, j % SLOTS).wait()


_mesh = plsc.VectorSubcoreMesh(
    core_axis_name="c", subcore_axis_name="s", num_cores=NC, num_subcores=NS)


@functools.partial(
    pl.kernel,
    mesh=_mesh,
    compiler_params=pltpu.CompilerParams(needs_layout_passes=False),
    scratch_types=[
        pltpu.VMEM((B,), jnp.int32),
        pltpu.VMEM((N,), jnp.int32),
        pltpu.VMEM((B,), jnp.int32),
        pltpu.VMEM((B,), jnp.float32),
        pltpu.VMEM((B,), jnp.int32),
        pltpu.VMEM((B,), jnp.int32),
        pltpu.VMEM((B,), jnp.float32),
        pltpu.VMEM((B,), jnp.float32),
        pltpu.SemaphoreType.DMA,
    ],
)
def _sc_scatter(idx_hbm, v_hbm, out_hbm,
                idx_v, m_v, winj_v, vrow_v, addr0, addr1, val0, val1, sem):
    addr_b = (addr0, addr1)
    val_b = (val0, val1)
    wid = lax.axis_index("s") * NC + lax.axis_index("c")
    pltpu.sync_copy(idx_hbm, idx_v)
    lane = lax.iota(jnp.int32, L)

    def _mb(k, _):
        idxc = idx_v[pl.ds(k * L, L)]
        jvec = k * L + lane
        plsc.store_scatter(m_v, [idxc], jvec)

        def _fix(i, __):
            w = plsc.load_gather(m_v, [idxc])
            plsc.store_scatter(m_v, [idxc], jvec, mask=jvec > w)
            return 0

        lax.fori_loop(0, 15, _fix, 0)
        return 0

    lax.fori_loop(0, B // L, _mb, 0)

    def _wj(k, _):
        idxc = idx_v[pl.ds(k * L, L)]
        winj_v[pl.ds(k * L, L)] = plsc.load_gather(m_v, [idxc])
        return 0

    lax.fori_loop(0, B // L, _wj, 0)

    def _stage(r, p):
        rg = wid * RPW + r
        pltpu.sync_copy(v_hbm.at[rg], vrow_v)
        base = rg * N

        def _chunk(k, _):
            s = k * L
            idxc = idx_v[pl.ds(s, L)]
            addr_b[p][pl.ds(s, L)] = idxc + base
            w = winj_v[pl.ds(s, L)]
            val_b[p][pl.ds(s, L)] = plsc.load_gather(vrow_v, [w])
            return 0

        lax.fori_loop(0, B // L, _chunk, 0)

    def _fire(p):
        return pltpu.async_copy(val_b[p], out_hbm.at[addr_b[p]], sem)

    _stage(0, 0)
    h = _fire(0)
    for r in range(1, RPW):
        p = r % 2
        _stage(r, p)
        h.wait()
        h = _fire(p)
    h.wait()


def kernel(t, idx, v):
    t1 = t.reshape(R * N)
    v2 = v.reshape(R, B)

    out0 = pl.pallas_call(
        _copy_body,
        out_shape=jax.ShapeDtypeStruct((R * N,), jnp.float32),
        in_specs=[_HBM],
        out_specs=_HBM,
        scratch_shapes=[
            pltpu.VMEM((SLOTS, CH), jnp.float32),
            pltpu.SemaphoreType.DMA((SLOTS,)),
            pltpu.SemaphoreType.DMA((SLOTS,)),
        ],
    )(t1)

    ref = jax.new_ref(out0)
    _sc_scatter(idx, v2, ref)
    return ref[...].reshape(t.shape)
